# Initial kernel scaffold; baseline (speedup 1.0000x reference)
import jax
import jax.numpy as jnp
from jax.experimental import pallas as pl


def kernel(probas, labels):
    raise NotImplementedError("write your pallas kernel here")


# speedup vs baseline: 46.3148x; 46.3148x over previous
import functools

import jax
import jax.numpy as jnp
from jax import lax
from jax.experimental import pallas as pl
from jax.experimental.pallas import tpu as pltpu
from jax.experimental.pallas import tpu_sc as plsc

C = 19
NPIX = 4 * 512 * 512
PLANE = 512 * 512
NC = 2
NS = 16
NW = NC * NS
PPT = NPIX // NW
B_BINS = 1024
CH = 2048
HROWS = 2 * C


def _sc_histogram(probas3, labels2):
    mesh = plsc.VectorSubcoreMesh(core_axis_name="c", subcore_axis_name="s")

    @functools.partial(
        pl.kernel,
        out_type=jax.ShapeDtypeStruct((NW, HROWS * B_BINS), jnp.float32),
        mesh=mesh,
        compiler_params=pltpu.CompilerParams(needs_layout_passes=False),
        scratch_types=[
            pltpu.VMEM((HROWS * B_BINS,), jnp.float32),
            pltpu.VMEM((C, CH), jnp.float32),
            pltpu.VMEM((CH,), jnp.int32),
        ],
    )
    def body(probas_hbm, labels_hbm, out_hbm, hist, pbuf, lbuf):
        cid = lax.axis_index("c")
        sid = lax.axis_index("s")
        wid = sid * NC + cid
        batch = wid // (PLANE // PPT)
        off0 = (wid % (PLANE // PPT)) * PPT

        zeros16 = jnp.zeros((16,), jnp.float32)

        def zbody(i, carry):
            hist[pl.ds(i * 16, 16)] = zeros16
            return carry

        lax.fori_loop(0, (HROWS * B_BINS) // 16, zbody, 0)

        ones16 = jnp.full((16,), 1.0, jnp.float32)
        binf = jnp.float32(B_BINS)

        def chunk_body(ch, carry):
            off = off0 + ch * CH
            pltpu.sync_copy(probas_hbm.at[batch, :, pl.ds(off, CH)], pbuf)
            pltpu.sync_copy(labels_hbm.at[batch, pl.ds(off, CH)], lbuf)

            def vbody(v, c2):
                base = v * 16
                lb = lbuf[pl.ds(base, 16)]
                for c in range(C):
                    pv = pbuf[c, pl.ds(base, 16)]
                    is_pos = lb == c
                    ev = jnp.where(is_pos, 1.0 - pv, pv)
                    t = jnp.minimum((ev * binf).astype(jnp.int32), B_BINS - 1)
                    idx = (c * B_BINS + (B_BINS - 1)) - t
                    plsc.addupdate_scatter(hist, [idx], ones16)
                    plsc.addupdate_scatter(
                        hist, [idx + C * B_BINS], ones16, mask=is_pos
                    )
                return c2

            lax.fori_loop(0, CH // 16, vbody, 0)
            return carry

        lax.fori_loop(0, PPT // CH, chunk_body, 0)
        pltpu.sync_copy(hist, out_hbm.at[wid])

    return body(probas3, labels2)


def _stage2(hists3):
    def body(h_ref, out_ref):
        h = jnp.sum(h_ref[...], axis=0)
        cnt = h[0:C]
        pos = h[C : 2 * C]
        row = lax.broadcasted_iota(jnp.int32, (B_BINS, B_BINS), 0)
        col = lax.broadcasted_iota(jnp.int32, (B_BINS, B_BINS), 1)
        tri = (row <= col).astype(jnp.float32)
        n = jnp.dot(cnt, tri, preferred_element_type=jnp.float32)
        s = jnp.dot(pos, tri, preferred_element_type=jnp.float32)
        g = s[:, B_BINS - 1 : B_BINS]
        den = jnp.maximum(g + n - s, 0.5)
        jac = n / den
        last = jac[:, B_BINS - 1 : B_BINS]
        losses = (jnp.sum(jac, axis=1, keepdims=True) - 0.5 * last) / B_BINS
        pres = (g > 0).astype(jnp.float32)
        tot = jnp.sum(losses * pres)
        npres = jnp.sum(pres)
        out_ref[0, 0] = jnp.where(npres > 0, tot / npres, 0.0)

    return pl.pallas_call(
        body,
        out_shape=jax.ShapeDtypeStruct((1, 1), jnp.float32),
        out_specs=pl.BlockSpec(memory_space=pltpu.SMEM),
    )(hists3)


def kernel(probas, labels):
    probas3 = probas.reshape(4, C, PLANE)
    labels2 = labels.reshape(4, PLANE).astype(jnp.int32)
    hists = _sc_histogram(probas3, labels2)
    out = _stage2(hists.reshape(NW, HROWS, B_BINS))
    return out[0, 0]


# speedup vs baseline: 119.7082x; 2.5847x over previous
import functools

import jax
import jax.numpy as jnp
from jax import lax
from jax.experimental import pallas as pl
from jax.experimental.pallas import tpu as pltpu
from jax.experimental.pallas import tpu_sc as plsc

C = 19
NPIX = 4 * 512 * 512
PLANE = 512 * 512
NC = 2
NS = 16
NW = NC * NS
PPT = NPIX // NW
B_BINS = 1024
CH = 1024
HROWS = 2 * C


def _sc_histogram(probas3, labels2):
    mesh = plsc.VectorSubcoreMesh(core_axis_name="c", subcore_axis_name="s")
    nch = PPT // CH

    @functools.partial(
        pl.kernel,
        out_type=jax.ShapeDtypeStruct((NW, HROWS * B_BINS), jnp.float32),
        mesh=mesh,
        compiler_params=pltpu.CompilerParams(needs_layout_passes=False),
        scratch_types=[
            pltpu.VMEM((HROWS * B_BINS,), jnp.float32),
            pltpu.VMEM((C, CH), jnp.float32),
            pltpu.VMEM((C, CH), jnp.float32),
            pltpu.VMEM((PPT,), jnp.int32),
            pltpu.SemaphoreType.DMA,
            pltpu.SemaphoreType.DMA,
            pltpu.SemaphoreType.DMA,
        ],
    )
    def body(probas_hbm, labels_hbm, out_hbm, hist, pbuf0, pbuf1, lbuf,
             sem0, sem1, seml):
        cid = lax.axis_index("c")
        sid = lax.axis_index("s")
        wid = sid * NC + cid
        batch = wid // (PLANE // PPT)
        off0 = (wid % (PLANE // PPT)) * PPT

        pltpu.async_copy(labels_hbm.at[batch, pl.ds(off0, PPT)], lbuf, seml)
        pltpu.async_copy(probas_hbm.at[batch, :, pl.ds(off0, CH)], pbuf0, sem0)

        zeros16 = jnp.zeros((16,), jnp.float32)

        def zbody(i, carry):
            for k in range(8):
                hist[pl.ds(i * 128 + k * 16, 16)] = zeros16
            return carry

        lax.fori_loop(0, (HROWS * B_BINS) // 128, zbody, 0)

        pltpu.make_async_copy(labels_hbm.at[batch, pl.ds(off0, PPT)], lbuf,
                              seml).wait()

        ones16 = jnp.full((16,), 1.0, jnp.float32)
        mones16 = jnp.full((16,), -1.0, jnp.float32)
        binf = jnp.float32(B_BINS)
        iota16 = lax.iota(jnp.int32, 16)

        def compute(pbuf, ch):
            def vbody(v, c2):
                base = v * 16
                pvs = [pbuf[c, pl.ds(base, 16)] for c in range(C)]
                ts = [(pv * binf).astype(jnp.int32) for pv in pvs]
                ts = [jnp.minimum(t, B_BINS - 1) for t in ts]
                idxs = [
                    (c * B_BINS + (B_BINS - 1)) - ts[c] for c in range(C)
                ]
                for c in range(C):
                    plsc.addupdate_scatter(hist, [idxs[c]], ones16)
                lb = lbuf[pl.ds(ch * CH + base, 16)]
                col = iota16 + base
                pv = plsc.load_gather(pbuf, [lb, col])
                rowb = lb * B_BINS
                tw = jnp.minimum((pv * binf).astype(jnp.int32), B_BINS - 1)
                idx_wrong = rowb + (B_BINS - 1) - tw
                plsc.addupdate_scatter(hist, [idx_wrong], mones16)
                ev = 1.0 - pv
                tr = jnp.minimum((ev * binf).astype(jnp.int32), B_BINS - 1)
                idx_right = rowb + (B_BINS - 1) - tr
                plsc.addupdate_scatter(hist, [idx_right], ones16)
                plsc.addupdate_scatter(hist, [idx_right + C * B_BINS], ones16)
                return c2

            lax.fori_loop(0, CH // 16, vbody, 0)

        def pair_body(p, carry):
            off = off0 + (2 * p) * CH
            nxt = off + CH
            pltpu.async_copy(probas_hbm.at[batch, :, pl.ds(nxt, CH)], pbuf1,
                             sem1)
            pltpu.make_async_copy(probas_hbm.at[batch, :, pl.ds(off, CH)],
                                  pbuf0, sem0).wait()
            compute(pbuf0, 2 * p)
            nxt2 = jnp.minimum(off + 2 * CH, off0 + (nch - 1) * CH)
            pltpu.async_copy(probas_hbm.at[batch, :, pl.ds(nxt2, CH)], pbuf0,
                             sem0)
            pltpu.make_async_copy(probas_hbm.at[batch, :, pl.ds(nxt, CH)],
                                  pbuf1, sem1).wait()
            compute(pbuf1, 2 * p + 1)
            return carry

        lax.fori_loop(0, nch // 2, pair_body, 0)
        pltpu.make_async_copy(
            probas_hbm.at[batch, :, pl.ds(off0 + (nch - 1) * CH, CH)], pbuf0,
            sem0).wait()
        pltpu.sync_copy(hist, out_hbm.at[wid])

    return body(probas3, labels2)


def _stage2(hists3):
    def body(h_ref, out_ref):
        h = jnp.sum(h_ref[...], axis=0)
        cnt = h[0:C]
        pos = h[C : 2 * C]
        row = lax.broadcasted_iota(jnp.int32, (B_BINS, B_BINS), 0)
        col = lax.broadcasted_iota(jnp.int32, (B_BINS, B_BINS), 1)
        tri = (row <= col).astype(jnp.float32)
        n = jnp.dot(cnt, tri, preferred_element_type=jnp.float32)
        s = jnp.dot(pos, tri, preferred_element_type=jnp.float32)
        g = s[:, B_BINS - 1 : B_BINS]
        den = jnp.maximum(g + n - s, 0.5)
        jac = n / den
        last = jac[:, B_BINS - 1 : B_BINS]
        losses = (jnp.sum(jac, axis=1, keepdims=True) - 0.5 * last) / B_BINS
        pres = (g > 0).astype(jnp.float32)
        tot = jnp.sum(losses * pres)
        npres = jnp.sum(pres)
        out_ref[0, 0] = jnp.where(npres > 0, tot / npres, 0.0)

    return pl.pallas_call(
        body,
        out_shape=jax.ShapeDtypeStruct((1, 1), jnp.float32),
        out_specs=pl.BlockSpec(memory_space=pltpu.SMEM),
    )(hists3)


def kernel(probas, labels):
    probas3 = probas.reshape(4, C, PLANE)
    labels2 = labels.reshape(4, PLANE).astype(jnp.int32)
    hists = _sc_histogram(probas3, labels2)
    out = _stage2(hists.reshape(NW, HROWS, B_BINS))
    return out[0, 0]


# speedup vs baseline: 125.5094x; 1.0485x over previous
import functools

import jax
import jax.numpy as jnp
from jax import lax
from jax.experimental import pallas as pl
from jax.experimental.pallas import tpu as pltpu
from jax.experimental.pallas import tpu_sc as plsc

C = 19
NPIX = 4 * 512 * 512
PLANE = 512 * 512
NC = 2
NS = 16
NW = NC * NS
PPT = NPIX // NW
B_BINS = 1024
CH = 1024
HROWS = 2 * C


def _sc_histogram(probas3, labels2):
    mesh = plsc.VectorSubcoreMesh(core_axis_name="c", subcore_axis_name="s")
    nch = PPT // CH

    @functools.partial(
        pl.kernel,
        out_type=jax.ShapeDtypeStruct((NW, HROWS * B_BINS), jnp.float32),
        mesh=mesh,
        compiler_params=pltpu.CompilerParams(needs_layout_passes=False),
        scratch_types=[
            pltpu.VMEM((HROWS * B_BINS,), jnp.float32),
            pltpu.VMEM((C, CH), jnp.float32),
            pltpu.VMEM((C, CH), jnp.float32),
            pltpu.VMEM((PPT,), jnp.int32),
            pltpu.SemaphoreType.DMA,
            pltpu.SemaphoreType.DMA,
            pltpu.SemaphoreType.DMA,
        ],
    )
    def body(probas_hbm, labels_hbm, out_hbm, hist, pbuf0, pbuf1, lbuf,
             sem0, sem1, seml):
        cid = lax.axis_index("c")
        sid = lax.axis_index("s")
        wid = sid * NC + cid
        batch = wid // (PLANE // PPT)
        off0 = (wid % (PLANE // PPT)) * PPT

        pltpu.async_copy(labels_hbm.at[batch, pl.ds(off0, PPT)], lbuf, seml)
        pltpu.async_copy(probas_hbm.at[batch, :, pl.ds(off0, CH)], pbuf0, sem0)

        zeros16 = jnp.zeros((16,), jnp.float32)

        def zbody(i, carry):
            for k in range(8):
                hist[pl.ds(i * 128 + k * 16, 16)] = zeros16
            return carry

        lax.fori_loop(0, (HROWS * B_BINS) // 128, zbody, 0)

        pltpu.make_async_copy(labels_hbm.at[batch, pl.ds(off0, PPT)], lbuf,
                              seml).wait()

        ones16 = jnp.full((16,), 1.0, jnp.float32)
        mones16 = jnp.full((16,), -1.0, jnp.float32)
        binf = jnp.float32(B_BINS - 0.01)
        iota16 = lax.iota(jnp.int32, 16)

        def group(pbuf, ch, v, base):
            pvs = [pbuf[c, pl.ds(base, 16)] for c in range(C)]
            ts = [(pv * binf).astype(jnp.int32) for pv in pvs]
            idxs = [(c * B_BINS + (B_BINS - 1)) - ts[c] for c in range(C)]
            for c in range(C):
                plsc.addupdate_scatter(hist, [idxs[c]], ones16)
            lb = lbuf[pl.ds(ch * CH + base, 16)]
            col = iota16 + base
            pv = plsc.load_gather(pbuf, [lb, col])
            rowb = lb * B_BINS
            tw = (pv * binf).astype(jnp.int32)
            idx_wrong = rowb + (B_BINS - 1) - tw
            plsc.addupdate_scatter(hist, [idx_wrong], mones16)
            idx_right = rowb + tw
            plsc.addupdate_scatter(hist, [idx_right], ones16)
            plsc.addupdate_scatter(hist, [idx_right + C * B_BINS], ones16)

        def compute(pbuf, ch):
            def vbody(v, c2):
                group(pbuf, ch, v, v * 32)
                group(pbuf, ch, v, v * 32 + 16)
                return c2

            lax.fori_loop(0, CH // 32, vbody, 0)

        def pair_body(p, carry):
            off = off0 + (2 * p) * CH
            nxt = off + CH
            pltpu.async_copy(probas_hbm.at[batch, :, pl.ds(nxt, CH)], pbuf1,
                             sem1)
            pltpu.make_async_copy(probas_hbm.at[batch, :, pl.ds(off, CH)],
                                  pbuf0, sem0).wait()
            compute(pbuf0, 2 * p)
            nxt2 = jnp.minimum(off + 2 * CH, off0 + (nch - 1) * CH)
            pltpu.async_copy(probas_hbm.at[batch, :, pl.ds(nxt2, CH)], pbuf0,
                             sem0)
            pltpu.make_async_copy(probas_hbm.at[batch, :, pl.ds(nxt, CH)],
                                  pbuf1, sem1).wait()
            compute(pbuf1, 2 * p + 1)
            return carry

        lax.fori_loop(0, nch // 2, pair_body, 0)
        pltpu.make_async_copy(
            probas_hbm.at[batch, :, pl.ds(off0 + (nch - 1) * CH, CH)], pbuf0,
            sem0).wait()
        pltpu.sync_copy(hist, out_hbm.at[wid])

    return body(probas3, labels2)


def _stage2(hists3):
    def body(h_ref, out_ref):
        h = jnp.sum(h_ref[...], axis=0)
        cnt = h[0:C]
        pos = h[C : 2 * C]
        row = lax.broadcasted_iota(jnp.int32, (B_BINS, B_BINS), 0)
        col = lax.broadcasted_iota(jnp.int32, (B_BINS, B_BINS), 1)
        tri = (row <= col).astype(jnp.float32)
        n = jnp.dot(cnt, tri, preferred_element_type=jnp.float32)
        s = jnp.dot(pos, tri, preferred_element_type=jnp.float32)
        g = s[:, B_BINS - 1 : B_BINS]
        den = jnp.maximum(g + n - s, 0.5)
        jac = n / den
        last = jac[:, B_BINS - 1 : B_BINS]
        losses = (jnp.sum(jac, axis=1, keepdims=True) - 0.5 * last) / B_BINS
        pres = (g > 0).astype(jnp.float32)
        tot = jnp.sum(losses * pres)
        npres = jnp.sum(pres)
        out_ref[0, 0] = jnp.where(npres > 0, tot / npres, 0.0)

    return pl.pallas_call(
        body,
        out_shape=jax.ShapeDtypeStruct((1, 1), jnp.float32),
        out_specs=pl.BlockSpec(memory_space=pltpu.SMEM),
    )(hists3)


def kernel(probas, labels):
    probas3 = probas.reshape(4, C, PLANE)
    labels2 = labels.reshape(4, PLANE).astype(jnp.int32)
    hists = _sc_histogram(probas3, labels2)
    out = _stage2(hists.reshape(NW, HROWS, B_BINS))
    return out[0, 0]


# speedup vs baseline: 295.0738x; 2.3510x over previous
import functools

import jax
import jax.numpy as jnp
from jax import lax
from jax.experimental import pallas as pl
from jax.experimental.pallas import tpu as pltpu
from jax.experimental.pallas import tpu_sc as plsc

C = 19
NPIX = 4 * 512 * 512
PLANE = 512 * 512
NC = 2
NS = 16
NW = NC * NS
PPT = NPIX // NW
B_BINS = 1024
CH = 1024
HROWS = 2 * C
EXP_OFF = 0x1FC00
PAD = 128
HSIZE = PAD + HROWS * B_BINS + 128


def _sc_histogram(probas4, labels3):
    mesh = plsc.VectorSubcoreMesh(core_axis_name="c", subcore_axis_name="s")
    nch = PPT // CH
    W = 512
    RW = CH // W
    RPT = PPT // W
    TPB = PLANE // PPT

    @functools.partial(
        pl.kernel,
        out_type=jax.ShapeDtypeStruct((NW, HROWS, B_BINS), jnp.float32),
        mesh=mesh,
        compiler_params=pltpu.CompilerParams(needs_layout_passes=False),
        scratch_types=[
            pltpu.VMEM((HSIZE,), jnp.float32),
            pltpu.VMEM((C, RW, W), jnp.float32),
            pltpu.VMEM((C, RW, W), jnp.float32),
            pltpu.VMEM((RPT, W), jnp.int32),
            pltpu.SemaphoreType.DMA,
            pltpu.SemaphoreType.DMA,
            pltpu.SemaphoreType.DMA,
        ],
    )
    def body(probas_hbm, labels_hbm, out_hbm, hist, pbuf0, pbuf1, lbuf,
             sem0, sem1, seml):
        cid = lax.axis_index("c")
        sid = lax.axis_index("s")
        wid = sid * NC + cid
        batch = wid // TPB
        row0 = (wid % TPB) * RPT

        pltpu.async_copy(labels_hbm.at[batch, pl.ds(row0, RPT), :], lbuf, seml)
        pltpu.async_copy(probas_hbm.at[batch, :, pl.ds(row0, RW), :], pbuf0,
                         sem0)

        zeros16 = jnp.zeros((16,), jnp.float32)

        @plsc.parallel_loop(0, HSIZE // 16, unroll=4)
        def _zero(i):
            hist[pl.ds(i * 16, 16)] = zeros16

        pltpu.make_async_copy(labels_hbm.at[batch, pl.ds(row0, RPT), :], lbuf,
                              seml).wait()

        ones16 = jnp.full((16,), 1.0, jnp.float32)
        one_f = jnp.float32(1.0)
        iota16 = lax.iota(jnp.int32, 16)

        def bin_bits(pv):
            return lax.shift_right_logical(
                plsc.bitcast(pv + one_f, jnp.int32), 13
            )

        def group(pbuf, ch, r, cb):
            pvs = [pbuf[c, r, pl.ds(cb, 16)] for c in range(C)]
            us = [bin_bits(pv) for pv in pvs]
            idxs = [
                (PAD + c * B_BINS + (B_BINS - 1) + EXP_OFF) - us[c]
                for c in range(C)
            ]
            for c in range(C):
                plsc.addupdate_scatter(hist, [idxs[c]], ones16)
            lb = lbuf[ch * RW + r, pl.ds(cb, 16)]
            rv = iota16 * 0 + r
            colv = iota16 + cb
            pv = plsc.load_gather(pbuf, [lb, rv, colv])
            rowb = lb * B_BINS
            u = bin_bits(pv)
            idx_pos = rowb + (u + (PAD - EXP_OFF + C * B_BINS))
            plsc.addupdate_scatter(hist, [idx_pos], ones16)

        def compute(pbuf, ch):
            @plsc.parallel_loop(0, CH // 16, unroll=2)
            def _vbody(v):
                r = v // (W // 16)
                cb = (v % (W // 16)) * 16
                group(pbuf, ch, r, cb)

        def pair_body(p, carry):
            r_cur = row0 + (2 * p) * RW
            r_nxt = r_cur + RW
            pltpu.async_copy(probas_hbm.at[batch, :, pl.ds(r_nxt, RW), :],
                             pbuf1, sem1)
            pltpu.make_async_copy(probas_hbm.at[batch, :, pl.ds(r_cur, RW), :],
                                  pbuf0, sem0).wait()
            compute(pbuf0, 2 * p)
            r_nxt2 = jnp.minimum(r_cur + 2 * RW, row0 + (nch - 1) * RW)
            pltpu.async_copy(probas_hbm.at[batch, :, pl.ds(r_nxt2, RW), :],
                             pbuf0, sem0)
            pltpu.make_async_copy(probas_hbm.at[batch, :, pl.ds(r_nxt, RW), :],
                                  pbuf1, sem1).wait()
            compute(pbuf1, 2 * p + 1)
            return carry

        lax.fori_loop(0, nch // 2, pair_body, 0)
        pltpu.make_async_copy(
            probas_hbm.at[batch, :, pl.ds(row0 + (nch - 1) * RW, RW), :],
            pbuf0, sem0).wait()
        for r in range(HROWS):
            pltpu.sync_copy(hist.at[pl.ds(PAD + r * B_BINS, B_BINS)],
                            out_hbm.at[wid, r])

    return body(probas4, labels3)


def _stage2(hists3):
    def body(h_ref, out_ref):
        h = jnp.sum(h_ref[...], axis=0)
        cnt = h[0:C]
        pos = h[C : 2 * C]
        row = lax.broadcasted_iota(jnp.int32, (B_BINS, B_BINS), 0)
        col = lax.broadcasted_iota(jnp.int32, (B_BINS, B_BINS), 1)
        tri = (row <= col).astype(jnp.float32)
        rtri = ((B_BINS - 1 - row) <= col).astype(jnp.float32)
        n = jnp.dot(cnt + pos, tri, preferred_element_type=jnp.float32) - \
            jnp.dot(pos, rtri, preferred_element_type=jnp.float32)
        s = jnp.dot(pos, tri, preferred_element_type=jnp.float32)
        g = s[:, B_BINS - 1 : B_BINS]
        den = jnp.maximum(g + n - s, 0.5)
        jac = n / den
        last = jac[:, B_BINS - 1 : B_BINS]
        losses = (jnp.sum(jac, axis=1, keepdims=True) - 0.5 * last) / B_BINS
        pres = (g > 0).astype(jnp.float32)
        tot = jnp.sum(losses * pres)
        npres = jnp.sum(pres)
        out_ref[0, 0] = jnp.where(npres > 0, tot / npres, 0.0)

    return pl.pallas_call(
        body,
        out_shape=jax.ShapeDtypeStruct((1, 1), jnp.float32),
        out_specs=pl.BlockSpec(memory_space=pltpu.SMEM),
    )(hists3)


def kernel(probas, labels):
    hists = _sc_histogram(probas, labels.astype(jnp.int32))
    out = _stage2(hists)
    return out[0, 0]
